# baseline (speedup 1.0000x reference)
import functools

import jax
import jax.numpy as jnp
from jax import lax
from jax.experimental import pallas as pl
from jax.experimental.pallas import tpu as pltpu
from jax.experimental.pallas import tpu_sc as plsc

N = 3200000
NC = 2
NS = 16
NW = NC * NS
LANES = 16

MS = 1024000
CHUNK = MS // NW
NSUB = 5
SUB = CHUNK // NSUB
UNROLL = 5
SITERS = SUB // (UNROLL * LANES)

MT = N - MS
TROWS = MT // 128
TGRID = 5
TBLK = TROWS // TGRID

_mesh = plsc.VectorSubcoreMesh(core_axis_name="c", subcore_axis_name="s")


@functools.partial(
    pl.kernel,
    out_type=jax.ShapeDtypeStruct((NC * 8,), jnp.float32),
    mesh=_mesh,
    scratch_types=[
        [pltpu.VMEM((SUB,), jnp.float32) for _ in range(NSUB)],
        pltpu.VMEM((LANES,), jnp.float32),
        pltpu.VMEM((NS, LANES), jnp.float32),
        pltpu.VMEM_SHARED((NS, LANES), jnp.float32),
        [pltpu.SemaphoreType.DMA for _ in range(NSUB)],
    ],
)
def _partial_sums(x_hbm, out_hbm, bufs, part, gather_buf, shared, sems):
    c = lax.axis_index("c")
    s = lax.axis_index("s")
    wid = s * NC + c
    base = MT + wid * CHUNK

    copies = [
        pltpu.make_async_copy(
            x_hbm.at[pl.ds(base + k * SUB, SUB)], bufs[k], sems[k]
        )
        for k in range(NSUB)
    ]
    for k in range(NSUB):
        copies[k].start()

    total = jnp.zeros((LANES,), jnp.float32)
    for k in range(NSUB):
        copies[k].wait()

        def body(i, accs, buf=bufs[k]):
            off = i * (UNROLL * LANES)
            return tuple(
                accs[j] + buf[pl.ds(off + j * LANES, LANES)]
                for j in range(UNROLL)
            )

        zero = jnp.zeros((LANES,), jnp.float32)
        accs = lax.fori_loop(0, SITERS, body, (zero,) * UNROLL)
        for j in range(UNROLL):
            total = total + accs[j]

    part[...] = total
    pltpu.sync_copy(part, shared.at[s])
    plsc.subcore_barrier()

    @pl.when(s == 0)
    def _():
        pltpu.sync_copy(shared, gather_buf)
        core_total = gather_buf[0, :]
        for i in range(1, NS):
            core_total = core_total + gather_buf[i, :]
        scalar = core_total[0]
        for i in range(1, LANES):
            scalar = scalar + core_total[i]
        part[...] = jnp.full((LANES,), scalar, jnp.float32)
        pltpu.sync_copy(part.at[pl.ds(0, 8)], out_hbm.at[pl.ds(c * 8, 8)])


def _tc_reduce_body(x_ref, out_ref, acc_ref):
    i = pl.program_id(0)

    @pl.when(i == 0)
    def _():
        acc_ref[...] = jnp.zeros_like(acc_ref)

    ones = jnp.ones((8, TBLK), jnp.float32)
    acc_ref[...] += jax.lax.dot(
        ones, x_ref[...], precision=jax.lax.Precision.HIGHEST
    )

    @pl.when(i == TGRID - 1)
    def _():
        out_ref[...] = jnp.sum(acc_ref[0:1, :]).reshape(1, 1)


_tc_reduce = pl.pallas_call(
    _tc_reduce_body,
    grid=(TGRID,),
    in_specs=[pl.BlockSpec((TBLK, 128), lambda i: (i, 0))],
    out_specs=pl.BlockSpec((1, 1), lambda i: (0, 0)),
    out_shape=jax.ShapeDtypeStruct((1, 1), jnp.float32),
    scratch_shapes=[pltpu.VMEM((8, 128), jnp.float32)],
)


def kernel(atomic_energy):
    x = atomic_energy.reshape(-1)
    core_totals = _partial_sums(x)
    tc_total = _tc_reduce(x.reshape(N // 128, 128))
    return core_totals[0:1].reshape(1, 1) + core_totals[8:9].reshape(1, 1) + tc_total
